# Initial kernel scaffold; baseline (speedup 1.0000x reference)
import jax
import jax.numpy as jnp
from jax.experimental import pallas as pl


def kernel(word_ids, word_emb, W, b, ln_gamma, ln_beta):
    raise NotImplementedError("write your pallas kernel here")


# speedup vs baseline: 6.7852x; 6.7852x over previous
import functools
import math

import jax
import jax.numpy as jnp
from jax import lax
from jax.experimental import pallas as pl
from jax.experimental.pallas import tpu as pltpu
from jax.experimental.pallas import tpu_sc as plsc

_VOCAB = 100000
_EMB = 64
_B = 16384
_L = 20

_NC = 2
_NS = 16
_NW = _NC * _NS
_ROWS_PER_W = _B // _NW
_CHUNK = 64
_NCHUNK = _ROWS_PER_W // _CHUNK
_IDX_PER_CHUNK = _CHUNK * _L
_GATHER_N = 128
_NGATHER = _IDX_PER_CHUNK // _GATHER_N


def _sc_pool_body(table_hbm, ids_hbm, out_hbm, idx_v, rows_v, out_v, sem):
    wid = lax.axis_index("s") * _NC + lax.axis_index("c")
    lane = lax.iota(jnp.int32, 16)
    tail = lane >= 12

    for c in range(_NCHUNK):
        gbase = wid * _ROWS_PER_W + c * _CHUNK
        ibase = gbase * _L
        pltpu.sync_copy(ids_hbm.at[pl.ds(ibase, _IDX_PER_CHUNK)], idx_v)
        copies = [
            pltpu.async_copy(
                table_hbm.at[idx_v.at[pl.ds(j * _GATHER_N, _GATHER_N)]],
                rows_v.at[pl.ds(j * _GATHER_N, _GATHER_N)],
                sem,
            )
            for j in range(_NGATHER)
        ]
        for cp in copies:
            cp.wait()

        def body(b, carry):
            base = b * _L
            v1 = idx_v[pl.ds(base, 16)]
            v2 = idx_v[pl.ds(base + 4, 16)]
            c1 = plsc.all_reduce_population_count(v1 != 0)
            c2 = plsc.all_reduce_population_count(tail & (v2 != 0))
            cnt = jnp.maximum((c1 + c2).astype(jnp.float32), 1.0)
            inv = 1.0 / cnt
            for k in range(4):
                acc = rows_v[base, pl.ds(k * 16, 16)]
                for l in range(1, _L):
                    acc = acc + rows_v[base + l, pl.ds(k * 16, 16)]
                out_v[b, pl.ds(k * 16, 16)] = acc * inv
            return carry

        lax.fori_loop(0, _CHUNK, body, 0)
        pltpu.sync_copy(out_v, out_hbm.at[pl.ds(gbase, _CHUNK)])


_sc_pool = pl.kernel(
    _sc_pool_body,
    out_type=jax.ShapeDtypeStruct((_B, _EMB), jnp.float32),
    mesh=plsc.VectorSubcoreMesh(core_axis_name="c", subcore_axis_name="s"),
    compiler_params=pltpu.CompilerParams(
        needs_layout_passes=False, use_tc_tiling_on_sc=False
    ),
    scratch_types=[
        pltpu.VMEM((_IDX_PER_CHUNK,), jnp.int32),
        pltpu.VMEM((_IDX_PER_CHUNK, _EMB), jnp.float32),
        pltpu.VMEM((_CHUNK, _EMB), jnp.float32),
        pltpu.SemaphoreType.DMA,
    ],
)

_INV_SQRT2 = 1.0 / math.sqrt(2.0)


def _tc_head_body(x_ref, w_ref, b_ref, g_ref, beta_ref, o_ref):
    x = x_ref[...]
    h = lax.dot_general(
        x, w_ref[...], (((1,), (1,)), ((), ())),
        preferred_element_type=jnp.float32,
    )
    h = h + b_ref[...]
    mu = jnp.mean(h, axis=1, keepdims=True)
    d = h - mu
    var = jnp.mean(d * d, axis=1, keepdims=True)
    hn = d * lax.rsqrt(var + 1e-5) * g_ref[...] + beta_ref[...]
    o_ref[...] = 0.5 * hn * (1.0 + lax.erf(hn * _INV_SQRT2))


_TC_BLK = 1024


def _tc_head(x, W, b2, g2, beta2):
    grid = (_B // _TC_BLK,)
    return pl.pallas_call(
        _tc_head_body,
        grid=grid,
        in_specs=[
            pl.BlockSpec((_TC_BLK, _EMB), lambda i: (i, 0)),
            pl.BlockSpec((_EMB, _EMB), lambda i: (0, 0)),
            pl.BlockSpec((1, _EMB), lambda i: (0, 0)),
            pl.BlockSpec((1, _EMB), lambda i: (0, 0)),
            pl.BlockSpec((1, _EMB), lambda i: (0, 0)),
        ],
        out_specs=pl.BlockSpec((_TC_BLK, _EMB), lambda i: (i, 0)),
        out_shape=jax.ShapeDtypeStruct((_B, _EMB), jnp.float32),
    )(x, W, b2, g2, beta2)


def kernel(word_ids, word_emb, W, b, ln_gamma, ln_beta):
    ids = word_ids.reshape(-1).astype(jnp.int32)
    mean_emb = _sc_pool(word_emb, ids)
    return _tc_head(
        mean_emb,
        W,
        b.reshape(1, _EMB),
        ln_gamma.reshape(1, _EMB),
        ln_beta.reshape(1, _EMB),
    )
